# baseline (speedup 1.0000x reference)
import functools

import jax
import jax.numpy as jnp
from jax import lax
from jax.experimental import pallas as pl
from jax.experimental.pallas import tpu as pltpu
from jax.experimental.pallas import tpu_sc as plsc

_NC = 2
_NS = 16
_NW = _NC * _NS
_NBUF = 3


def _sc_gather_sum(table, idx3, K, NCH_EFF, NCHUNKS):
    R, HD = table.shape
    NCH_PAD, NW, CBK = idx3.shape
    CB = CBK // K

    mesh = plsc.VectorSubcoreMesh(core_axis_name="c", subcore_axis_name="s")

    @functools.partial(
        pl.kernel,
        out_type=jax.ShapeDtypeStruct((R * HD,), jnp.float32),
        mesh=mesh,
        compiler_params=pltpu.CompilerParams(
            needs_layout_passes=False, use_tc_tiling_on_sc=False),
        scratch_types=[
            pltpu.VMEM((NCH_PAD, CBK), jnp.int32),
            [pltpu.VMEM((CBK, HD), jnp.float32) for _ in range(_NBUF)],
            [pltpu.VMEM((CB * HD,), jnp.float32) for _ in range(_NBUF)],
            [pltpu.SemaphoreType.DMA for _ in range(_NBUF)],
            [pltpu.SemaphoreType.DMA for _ in range(_NBUF)],
        ],
    )
    def k(tab_hbm, idx_hbm, out_hbm, idx_v, gbufs, obufs, gsems, osems):
        wid = lax.axis_index("s") * _NC + lax.axis_index("c")
        pltpu.sync_copy(idx_hbm.at[:, wid], idx_v)

        def e0_of(c):
            return pl.multiple_of(
                jnp.minimum(c * NW + wid, NCHUNKS - 1) * (CB * HD), CB * HD)

        for b in range(_NBUF):
            pltpu.make_async_copy(
                tab_hbm.at[idx_v.at[b]], gbufs[b], gsems[b]).start()

        def step(i, carry):
            for b in range(_NBUF):
                c = i * _NBUF + b
                gb, ob = gbufs[b], obufs[b]
                gs, os_ = gsems[b], osems[b]
                e0 = e0_of(c)
                pltpu.make_async_copy(tab_hbm.at[idx_v.at[c]], gb, gs).wait()
                @pl.when(c >= _NBUF)
                def _():
                    pltpu.make_async_copy(
                        ob, out_hbm.at[pl.ds(0, CB * HD)], os_).wait()

                def row(r, rc):
                    rb = r * K
                    acc = [
                        plsc.bitcast(gb[rb, pl.ds(h * 16, 16)], jnp.bfloat16)
                        for h in range(HD // 16)
                    ]
                    for kk in range(1, K):
                        for h in range(HD // 16):
                            acc[h] += plsc.bitcast(
                                gb[rb + kk, pl.ds(h * 16, 16)], jnp.bfloat16)
                    for h in range(HD // 16):
                        ob[pl.ds(r * HD + h * 16, 16)] = plsc.bitcast(
                            acc[h], jnp.float32)
                    return rc

                lax.fori_loop(0, CB, row, 0)
                @pl.when(c + _NBUF < NCH_EFF)
                def _():
                    pltpu.make_async_copy(
                        tab_hbm.at[idx_v.at[c + _NBUF]], gb, gs).start()
                pltpu.make_async_copy(
                    ob, out_hbm.at[pl.ds(e0, CB * HD)], os_).start()
            return carry

        lax.fori_loop(0, NCH_EFF // _NBUF, step, 0)
        for b in range(_NBUF):
            pltpu.make_async_copy(
                obufs[b], out_hbm.at[pl.ds(0, CB * HD)], osems[b]).wait()

    return k(table, idx3)


def _tc_combine(Hf, G, W, b2, K):
    N, DHD = Hf.shape
    D, _, HD = G.shape
    BN = 1000
    scale = 1.0 / K

    def body(h_ref, g_ref, w_ref, b_ref, o_ref):
        w = w_ref[...]
        bb = b_ref[...]
        for d in range(D):
            g = g_ref[d].astype(jnp.float32) * scale
            m = lax.dot_general(g, w, (((1,), (1,)), ((), ())),
                                preferred_element_type=jnp.float32)
            o_ref[:, d * HD:(d + 1) * HD] = h_ref[:, d * HD:(d + 1) * HD] + m + bb

    return pl.pallas_call(
        body,
        grid=(N // BN,),
        in_specs=[
            pl.BlockSpec((BN, DHD), lambda i: (i, 0)),
            pl.BlockSpec((D, BN, HD), lambda i: (0, i, 0)),
            pl.BlockSpec((HD, HD), lambda i: (0, 0)),
            pl.BlockSpec((1, HD), lambda i: (0, 0)),
        ],
        out_specs=pl.BlockSpec((BN, DHD), lambda i: (i, 0)),
        out_shape=jax.ShapeDtypeStruct((N, DHD), jnp.float32),
    )(Hf, G, W, b2)


def kernel(H, knn_idx, W, b):
    N, D, HD = H.shape
    K = knn_idx.shape[-1]
    R = N * D
    CB = 8
    NCHUNKS = R // CB
    NCH_EFF = -(-NCHUNKS // _NW)
    NCH_EFF = -(-NCH_EFF // _NBUF) * _NBUF
    NCH_PAD = -(-(NCH_EFF) // 8) * 8

    HW = HD // 2
    H2 = lax.bitcast_convert_type(
        H.astype(jnp.bfloat16).reshape(R, HW, 2), jnp.float32)
    offs = jnp.arange(D, dtype=jnp.int32)[:, None, None]
    idx_chunks = (knn_idx * D + offs).reshape(NCHUNKS, CB * K)
    pad = NCH_PAD * _NW - NCHUNKS
    idx3 = jnp.concatenate(
        [idx_chunks,
         jnp.broadcast_to(idx_chunks[-1:], (pad, CB * K))],
        axis=0).reshape(NCH_PAD, _NW, CB * K)

    G = _sc_gather_sum(H2, idx3, K, NCH_EFF, NCHUNKS)
    Gb = lax.bitcast_convert_type(
        G.reshape(R, HW), jnp.bfloat16).reshape(D, N, HD)
    out = _tc_combine(H.reshape(N, D * HD), Gb,
                      W, b.reshape(1, HD), K)
    return out.reshape(N, D, HD)


# speedup vs baseline: 9.4621x; 9.4621x over previous
import functools

import jax
import jax.numpy as jnp
from jax import lax
from jax.experimental import pallas as pl
from jax.experimental.pallas import tpu as pltpu
from jax.experimental.pallas import tpu_sc as plsc

_NC = 2
_NS = 16
_NW = _NC * _NS
_NBUF = 3


def _tc_pack(Hf):
    R, HD = Hf.shape
    HW = HD // 2
    BR = 2000

    def body(h_ref, o_ref):
        u = lax.bitcast_convert_type(h_ref[...], jnp.uint32)
        r = u + jnp.uint32(0x7FFF) + ((u >> jnp.uint32(16)) & jnp.uint32(1))
        lo = r[:, :HW] >> jnp.uint32(16)
        hi = r[:, HW:] & jnp.uint32(0xFFFF0000)
        o_ref[...] = lax.bitcast_convert_type(lo | hi, jnp.float32)

    return pl.pallas_call(
        body,
        grid=(R // BR,),
        in_specs=[pl.BlockSpec((BR, HD), lambda i: (i, 0))],
        out_specs=pl.BlockSpec((BR, HW), lambda i: (i, 0)),
        out_shape=jax.ShapeDtypeStruct((R, HW), jnp.float32),
    )(Hf)


def _sc_gather_sum(table, idx3, K, NCH_EFF, NCHUNKS):
    R, HD = table.shape
    NCH_PAD, NW, CBK = idx3.shape
    CB = CBK // K

    mesh = plsc.VectorSubcoreMesh(core_axis_name="c", subcore_axis_name="s")

    @functools.partial(
        pl.kernel,
        out_type=jax.ShapeDtypeStruct((R * HD,), jnp.float32),
        mesh=mesh,
        compiler_params=pltpu.CompilerParams(
            needs_layout_passes=False, use_tc_tiling_on_sc=False),
        scratch_types=[
            pltpu.VMEM((NCH_PAD, CBK), jnp.int32),
            [pltpu.VMEM((CBK, HD), jnp.float32) for _ in range(_NBUF)],
            [pltpu.VMEM((CB * HD,), jnp.float32) for _ in range(_NBUF)],
            [pltpu.SemaphoreType.DMA for _ in range(_NBUF)],
            [pltpu.SemaphoreType.DMA for _ in range(_NBUF)],
        ],
    )
    def k(tab_hbm, idx_hbm, out_hbm, idx_v, gbufs, obufs, gsems, osems):
        wid = lax.axis_index("s") * _NC + lax.axis_index("c")
        pltpu.sync_copy(idx_hbm.at[:, wid], idx_v)

        def e0_of(c):
            return pl.multiple_of(
                jnp.minimum(c * NW + wid, NCHUNKS - 1) * (CB * HD), CB * HD)

        for b in range(_NBUF):
            pltpu.make_async_copy(
                tab_hbm.at[idx_v.at[b]], gbufs[b], gsems[b]).start()

        def step(i, carry):
            for b in range(_NBUF):
                c = i * _NBUF + b
                gb, ob = gbufs[b], obufs[b]
                gs, os_ = gsems[b], osems[b]
                e0 = e0_of(c)
                pltpu.make_async_copy(tab_hbm.at[idx_v.at[c]], gb, gs).wait()
                @pl.when(c >= _NBUF)
                def _():
                    pltpu.make_async_copy(
                        ob, out_hbm.at[pl.ds(0, CB * HD)], os_).wait()

                def row(r, rc):
                    rb = r * K
                    acc = [
                        plsc.bitcast(gb[rb, pl.ds(h * 16, 16)], jnp.bfloat16)
                        for h in range(HD // 16)
                    ]
                    for kk in range(1, K):
                        for h in range(HD // 16):
                            acc[h] += plsc.bitcast(
                                gb[rb + kk, pl.ds(h * 16, 16)], jnp.bfloat16)
                    for h in range(HD // 16):
                        ob[pl.ds(r * HD + h * 16, 16)] = plsc.bitcast(
                            acc[h], jnp.float32)
                    return rc

                lax.fori_loop(0, CB, row, 0)
                @pl.when(c + _NBUF < NCH_EFF)
                def _():
                    pltpu.make_async_copy(
                        tab_hbm.at[idx_v.at[c + _NBUF]], gb, gs).start()
                pltpu.make_async_copy(
                    ob, out_hbm.at[pl.ds(e0, CB * HD)], os_).start()
            return carry

        lax.fori_loop(0, NCH_EFF // _NBUF, step, 0)
        for b in range(_NBUF):
            pltpu.make_async_copy(
                obufs[b], out_hbm.at[pl.ds(0, CB * HD)], osems[b]).wait()

    return k(table, idx3)


def _tc_combine(Hf, G, W, b2, K):
    N, DHD = Hf.shape
    D, _, HW = G.shape
    HD = 2 * HW
    BN = 1000
    scale = 1.0 / K

    def body(h_ref, g_ref, w_ref, b_ref, o_ref):
        wlo = w_ref[:, :HW]
        whi = w_ref[:, HW:]
        bb = b_ref[...]
        for d in range(D):
            u = lax.bitcast_convert_type(g_ref[d], jnp.uint32)
            glo = lax.bitcast_convert_type(u << jnp.uint32(16), jnp.float32)
            ghi = lax.bitcast_convert_type(
                u & jnp.uint32(0xFFFF0000), jnp.float32)
            m = lax.dot_general(glo, wlo, (((1,), (1,)), ((), ())),
                                preferred_element_type=jnp.float32)
            m += lax.dot_general(ghi, whi, (((1,), (1,)), ((), ())),
                                 preferred_element_type=jnp.float32)
            o_ref[:, d * HD:(d + 1) * HD] = (
                h_ref[:, d * HD:(d + 1) * HD] + m * scale + bb)

    return pl.pallas_call(
        body,
        grid=(N // BN,),
        in_specs=[
            pl.BlockSpec((BN, DHD), lambda i: (i, 0)),
            pl.BlockSpec((D, BN, HW), lambda i: (0, i, 0)),
            pl.BlockSpec((HD, HD), lambda i: (0, 0)),
            pl.BlockSpec((1, HD), lambda i: (0, 0)),
        ],
        out_specs=pl.BlockSpec((BN, DHD), lambda i: (i, 0)),
        out_shape=jax.ShapeDtypeStruct((N, DHD), jnp.float32),
    )(Hf, G, W, b2)


def kernel(H, knn_idx, W, b):
    N, D, HD = H.shape
    K = knn_idx.shape[-1]
    R = N * D
    HW = HD // 2
    CB = 8
    NCHUNKS = R // CB
    NCH_EFF = -(-NCHUNKS // _NW)
    NCH_EFF = -(-NCH_EFF // _NBUF) * _NBUF
    NCH_PAD = -(-(NCH_EFF) // 8) * 8

    Hp = _tc_pack(H.reshape(R, HD))
    offs = jnp.arange(D, dtype=jnp.int32)[:, None, None]
    idx_chunks = (knn_idx * D + offs).reshape(NCHUNKS, CB * K)
    pad = NCH_PAD * _NW - NCHUNKS
    idx3 = jnp.concatenate(
        [idx_chunks,
         jnp.broadcast_to(idx_chunks[-1:], (pad, CB * K))],
        axis=0).reshape(NCH_PAD, _NW, CB * K)

    G = _sc_gather_sum(Hp, idx3, K, NCH_EFF, NCHUNKS)
    out = _tc_combine(H.reshape(N, D * HD), G.reshape(D, N, HW),
                      W, b.reshape(1, HD), K)
    return out.reshape(N, D, HD)
